# Initial kernel scaffold; baseline (speedup 1.0000x reference)
import jax
import jax.numpy as jnp
from jax.experimental import pallas as pl


def kernel(cat, cont, tables, W_proj, mask_token):
    raise NotImplementedError("write your pallas kernel here")


# speedup vs baseline: 1.8634x; 1.8634x over previous
import functools

import jax
import jax.numpy as jnp
import numpy as np
from jax import lax
from jax.experimental import pallas as pl
from jax.experimental.pallas import tpu as pltpu
from jax.experimental.pallas import tpu_sc as plsc

B, L, C, F = 1024, 20, 26, 6
VOCAB, D, LFREQ = 100000, 32, 8
NT = B * L
NFIELD = C + F
NROWS = NT * NFIELD

NC, NS, LANES = 2, 16, 16
NW = NC * NS
TOK_PER_W = NT // NW
TOK_CHUNK = 32
NCHUNK = TOK_PER_W // TOK_CHUNK
CAT_PER_CHUNK = TOK_CHUNK * C
CONT_PER_CHUNK = TOK_CHUNK * F
GB = 64
KG = CAT_PER_CHUNK // GB
KC = CONT_PER_CHUNK // GB


def _cont_tc_kernel(cont_ref, ws_ref, wc_ref, mask_ref, freqs_ref, out_ref):
    freqs = freqs_ref[0]
    vals = cont_ref[...]
    ys = []
    for i in range(F):
        v = vals[:, i]
        m = jnp.isnan(v)
        clean = jnp.where(m, jnp.zeros_like(v), v)
        x = clean[:, None] * freqs[None, :]
        ws = ws_ref[i * LFREQ:(i + 1) * LFREQ, :]
        wc = wc_ref[i * LFREQ:(i + 1) * LFREQ, :]
        y = (jnp.dot(jnp.sin(x), ws, preferred_element_type=jnp.float32)
             + jnp.dot(jnp.cos(x), wc, preferred_element_type=jnp.float32))
        ys.append(jnp.where(m[:, None], mask_ref[0], y))
    out_ref[...] = jnp.concatenate(ys, axis=1)


def _cont_embs(cont2, W_proj, mask_token):
    Ws = jnp.transpose(W_proj[:, :, 0::2], (0, 2, 1)).reshape(F * LFREQ, D)
    Wc = jnp.transpose(W_proj[:, :, 1::2], (0, 2, 1)).reshape(F * LFREQ, D)
    Tb = 2048
    grid = NT // Tb
    return pl.pallas_call(
        _cont_tc_kernel,
        grid=(grid,),
        in_specs=[
            pl.BlockSpec((Tb, F), lambda i: (i, 0)),
            pl.BlockSpec((F * LFREQ, D), lambda i: (0, 0)),
            pl.BlockSpec((F * LFREQ, D), lambda i: (0, 0)),
            pl.BlockSpec((1, D), lambda i: (0, 0)),
            pl.BlockSpec((1, LFREQ), lambda i: (0, 0)),
        ],
        out_specs=pl.BlockSpec((Tb, F * D), lambda i: (i, 0)),
        out_shape=jax.ShapeDtypeStruct((NT, F * D), jnp.float32),
    )(cont2, Ws, Wc, mask_token[None],
      jnp.asarray((2.0 ** np.arange(LFREQ)) * np.pi, jnp.float32)[None])


def _sc_body(tables_hbm, cat_hbm, cont_hbm, out_hbm,
             idxraw, gidx, rows, contb, foff, dstb, cdstb, dstc, cdst,
             gsem, ssem):
    wid = lax.axis_index("s") * NC + lax.axis_index("c")
    tok0 = wid * TOK_PER_W
    cat_base = tok0 * C
    lane = lax.iota(jnp.int32, LANES)

    for j in range(KG):
        for s in range(GB // LANES):
            p = j * GB + s * LANES + lane
            fld = p % C
            tokl = (p * 2521) >> 16
            sl = pl.ds(s * LANES, LANES)
            foff[j, sl] = fld * VOCAB
            dstb[j, sl] = (tok0 + tokl) * NFIELD + fld
    for j in range(KC):
        for s in range(GB // LANES):
            q = j * GB + s * LANES + lane
            qt = (q * 10923) >> 16
            sl = pl.ds(s * LANES, LANES)
            cdstb[j, sl] = (tok0 + qt) * NFIELD + C + q % F

    def chunk(g, _):
        dshift = g * (TOK_CHUNK * NFIELD)
        pltpu.sync_copy(cat_hbm.at[pl.ds(cat_base + g * CAT_PER_CHUNK,
                                         CAT_PER_CHUNK)], idxraw)
        for j in range(KG):
            for s in range(GB // LANES):
                sl = pl.ds(s * LANES, LANES)
                gidx[j, sl] = idxraw[pl.ds(j * GB + s * LANES, LANES)] + foff[j, sl]
                dstc[j, sl] = dstb[j, sl] + dshift
        gathers = [
            pltpu.async_copy(tables_hbm.at[gidx.at[j]],
                             rows.at[pl.ds(j * GB, GB)], gsem)
            for j in range(KG)
        ]
        pltpu.sync_copy(cont_hbm.at[pl.ds((tok0 + g * TOK_CHUNK) * F,
                                          CONT_PER_CHUNK)], contb)
        for j in range(KC):
            for s in range(GB // LANES):
                sl = pl.ds(s * LANES, LANES)
                cdst[j, sl] = cdstb[j, sl] + dshift
        for cp in gathers:
            cp.wait()
        scatters = [
            pltpu.async_copy(rows.at[pl.ds(j * GB, GB)],
                             out_hbm.at[dstc.at[j]], ssem)
            for j in range(KG)
        ] + [
            pltpu.async_copy(contb.at[pl.ds(j * GB, GB)],
                             out_hbm.at[cdst.at[j]], ssem)
            for j in range(KC)
        ]
        for cp in scatters:
            cp.wait()
        return 0

    lax.fori_loop(0, NCHUNK, chunk, 0)


@functools.partial(jax.jit, donate_argnums=())
def _run(cat_flat, cont_flat, tables_flat):
    mesh = plsc.VectorSubcoreMesh(core_axis_name="c", subcore_axis_name="s")
    sc = pl.kernel(
        _sc_body,
        out_type=jax.ShapeDtypeStruct((NROWS, D), jnp.float32),
        mesh=mesh,
        compiler_params=pltpu.CompilerParams(use_tc_tiling_on_sc=False),
        scratch_types=[
            pltpu.VMEM((CAT_PER_CHUNK,), jnp.int32),
            pltpu.VMEM((KG, GB), jnp.int32),
            pltpu.VMEM((CAT_PER_CHUNK, D), jnp.float32),
            pltpu.VMEM((CONT_PER_CHUNK, D), jnp.float32),
            pltpu.VMEM((KG, GB), jnp.int32),
            pltpu.VMEM((KG, GB), jnp.int32),
            pltpu.VMEM((KC, GB), jnp.int32),
            pltpu.VMEM((KG, GB), jnp.int32),
            pltpu.VMEM((KC, GB), jnp.int32),
            pltpu.SemaphoreType.DMA,
            pltpu.SemaphoreType.DMA,
        ],
    )
    return sc(tables_flat, cat_flat, cont_flat)


def kernel(cat, cont, tables, W_proj, mask_token):
    cat_flat = cat.astype(jnp.int32).reshape(NT * C)
    tables_flat = tables.reshape(C * VOCAB, D)
    cont_y = _cont_embs(cont.reshape(NT, F), W_proj, mask_token)
    out = _run(cat_flat, cont_y.reshape(NT * F, D), tables_flat)
    return out.reshape(NT, NFIELD, D)


# speedup vs baseline: 4.1953x; 2.2514x over previous
import functools

import jax
import jax.numpy as jnp
import numpy as np
from jax import lax
from jax.experimental import pallas as pl
from jax.experimental.pallas import tpu as pltpu
from jax.experimental.pallas import tpu_sc as plsc

B, L, C, F = 1024, 20, 26, 6
VOCAB, D, LFREQ = 100000, 32, 8
NT = B * L
NFIELD = C + F

NC, NS, LANES = 2, 16, 16
NW = NC * NS
TCHUNK = 2048
NCHUNK = NT // TCHUNK


def _cont_tc_kernel(cont_ref, ws_ref, wc_ref, mask_ref, freqs_ref, out_ref):
    freqs = freqs_ref[0]
    for i in range(F):
        v = cont_ref[i]
        m = jnp.isnan(v)
        clean = jnp.where(m, jnp.zeros_like(v), v)
        x = freqs[:, None] * clean[None, :]
        y = (jnp.dot(ws_ref[i], jnp.sin(x), preferred_element_type=jnp.float32)
             + jnp.dot(wc_ref[i], jnp.cos(x), preferred_element_type=jnp.float32))
        out_ref[i] = jnp.where(m[None, :], mask_ref[0][:, None], y)


def _cont_embs(cont2, W_proj, mask_token):
    Ws = W_proj[:, :, 0::2]
    Wc = W_proj[:, :, 1::2]
    return pl.pallas_call(
        _cont_tc_kernel,
        out_shape=jax.ShapeDtypeStruct((F, D, NT), jnp.float32),
    )(cont2, Ws, Wc, mask_token[None],
      jnp.asarray((2.0 ** np.arange(LFREQ)) * np.pi, jnp.float32)[None])


def _sc_body(tabs_hbm, cat2_hbm, conty_hbm, out_hbm,
             plane, outpl, idxb, gsem):
    d = lax.axis_index("s") * NC + lax.axis_index("c")

    def do_plane(k, _):
        @pl.when(k < C)
        def _():
            pltpu.sync_copy(tabs_hbm.at[k, d, :], plane)

            def chunk(ch, _):
                base = ch * TCHUNK
                pltpu.sync_copy(cat2_hbm.at[k, pl.ds(base, TCHUNK)], idxb)
                for i in range(TCHUNK // LANES):
                    iv = idxb[pl.ds(i * LANES, LANES)]
                    outpl[pl.ds(base + i * LANES, LANES)] = plsc.load_gather(
                        plane, [iv])
                return 0

            lax.fori_loop(0, NCHUNK, chunk, 0)
            pltpu.sync_copy(outpl, out_hbm.at[k, d, :])

        @pl.when(k >= C)
        def _():
            pltpu.sync_copy(conty_hbm.at[k - C, d, :], outpl)
            pltpu.sync_copy(outpl, out_hbm.at[k, d, :])

        return 0

    lax.fori_loop(0, NFIELD, do_plane, 0)


@jax.jit
def _run(tables_t, cat2, cont_y):
    mesh = plsc.VectorSubcoreMesh(core_axis_name="c", subcore_axis_name="s")
    sc = pl.kernel(
        _sc_body,
        out_type=jax.ShapeDtypeStruct((NFIELD, D, NT), jnp.float32),
        mesh=mesh,
        compiler_params=pltpu.CompilerParams(needs_layout_passes=False),
        scratch_types=[
            pltpu.VMEM((VOCAB,), jnp.float32),
            pltpu.VMEM((NT,), jnp.float32),
            pltpu.VMEM((TCHUNK,), jnp.int32),
            pltpu.SemaphoreType.DMA,
        ],
    )
    return sc(tables_t, cat2, cont_y)


def kernel(cat, cont, tables, W_proj, mask_token):
    tables_t = jnp.transpose(tables, (0, 2, 1))
    cat2 = jnp.transpose(cat.astype(jnp.int32).reshape(NT, C), (1, 0))
    cont_y = _cont_embs(
        jnp.transpose(cont.reshape(NT, F), (1, 0)), W_proj, mask_token)
    out_t = _run(tables_t, cat2, cont_y)
    return jnp.transpose(out_t, (2, 0, 1))
